# baseline (speedup 1.0000x reference)
import functools

import jax
import jax.numpy as jnp
from jax import lax
from jax.experimental import pallas as pl
from jax.experimental.pallas import tpu as pltpu
from jax.experimental.pallas import tpu_sc as plsc

IN_SIZE = 1024
HID = 1024
MEM_SIZE = 65536
MEM_DIM = 512
OUT_SIZE = 1024
TOP_K = 3

NUM_BLOCKS = 16
BLK = MEM_SIZE // NUM_BLOCKS

NC = 2
NS = 16
NW = NC * NS
LANES = 16
ROWS_PER_W = MEM_SIZE // NW
VECS_PER_W = ROWS_PER_W // LANES
CAND = TOP_K * LANES
NCAND = NW * CAND
NEG = -3.0e38
BIGI = 2**30


def _tc1_body(x_ref, W1_ref, b1_ref, W2_ref, b2_ref, Wq_ref, bq_ref,
              Wt_ref, bout_ref, mk_ref, imp_ref, sims_ref, pout_ref, qn_scr):
    i = pl.program_id(0)

    @pl.when(i == 0)
    def _():
        h1 = jnp.maximum(
            jnp.dot(x_ref[...], W1_ref[...], preferred_element_type=jnp.float32)
            + b1_ref[...], 0.0)
        h2 = jnp.dot(h1, W2_ref[...], preferred_element_type=jnp.float32) + b2_ref[...]
        q = jnp.dot(h2, Wq_ref[...], preferred_element_type=jnp.float32) + bq_ref[...]
        qn = q * (1.0 / jnp.maximum(jnp.sqrt(jnp.sum(q * q)), 1e-12))
        qn_scr[...] = qn
        pout_ref[...] = (jnp.dot(h2, Wt_ref[...], preferred_element_type=jnp.float32)
                         + bout_ref[...])

    mk = mk_ref[...]
    qn = qn_scr[...]
    dn = (((1,), (1,)), ((), ()))
    s = lax.dot_general(qn, mk, dn, preferred_element_type=jnp.float32)
    ones = jnp.ones((1, MEM_DIM), dtype=jnp.bfloat16)
    mk2 = (mk * mk).astype(jnp.bfloat16)
    rsq = lax.dot_general(ones, mk2, dn, preferred_element_type=jnp.float32)
    rn = 1.0 / jnp.maximum(jnp.sqrt(rsq), 1e-12)
    sims_ref[...] = (s * rn * imp_ref[...].reshape(1, BLK)).reshape(BLK)


def _tc1(x_last, W1, b1, W2, b2, Wq, bq, Wt, bout, mem_keys, imp):
    full2 = lambda shape: pl.BlockSpec(shape, lambda i: (0, 0))
    return pl.pallas_call(
        _tc1_body,
        grid=(NUM_BLOCKS,),
        in_specs=[
            full2((1, IN_SIZE)),
            full2((IN_SIZE, HID)),
            full2((1, HID)),
            full2((HID, HID)),
            full2((1, HID)),
            full2((HID, MEM_DIM)),
            full2((1, MEM_DIM)),
            full2((HID, OUT_SIZE)),
            full2((1, OUT_SIZE)),
            pl.BlockSpec((BLK, MEM_DIM), lambda i: (i, 0)),
            pl.BlockSpec((BLK,), lambda i: (i,)),
        ],
        out_specs=[
            pl.BlockSpec((BLK,), lambda i: (i,)),
            full2((1, OUT_SIZE)),
        ],
        out_shape=[
            jax.ShapeDtypeStruct((MEM_SIZE,), jnp.float32),
            jax.ShapeDtypeStruct((1, OUT_SIZE), jnp.float32),
        ],
        scratch_shapes=[pltpu.VMEM((1, MEM_DIM), jnp.float32)],
    )(x_last, W1, b1, W2, b2, Wq, bq, Wt, bout, mem_keys, imp)


def _tiecmp(v, i, mv, mi):
    return jnp.logical_or(v > mv, jnp.logical_and(v == mv, i < mi))


def _insert3(v, ii, carry):
    b1, b2, b3, i1, i2, i3 = carry
    c1 = _tiecmp(v, ii, b1, i1)
    nb1 = jnp.where(c1, v, b1)
    ni1 = jnp.where(c1, ii, i1)
    dv = jnp.where(c1, b1, v)
    di = jnp.where(c1, i1, ii)
    c2 = _tiecmp(dv, di, b2, i2)
    nb2 = jnp.where(c2, dv, b2)
    ni2 = jnp.where(c2, di, i2)
    dv2 = jnp.where(c2, b2, dv)
    di2 = jnp.where(c2, i2, di)
    c3 = _tiecmp(dv2, di2, b3, i3)
    nb3 = jnp.where(c3, dv2, b3)
    ni3 = jnp.where(c3, di2, i3)
    return (nb1, nb2, nb3, ni1, ni2, ni3)


def _permute(v, idx):
    dn = lax.GatherDimensionNumbers(
        offset_dims=(), collapsed_slice_dims=(0,), start_index_map=(0,))
    return lax.gather(v, idx[:, None], dn, slice_sizes=(1,),
                      mode=lax.GatherScatterMode.PROMISE_IN_BOUNDS)


NS1 = 16
ROWS1 = MEM_SIZE // NS1
VECS1 = ROWS1 // LANES
NCAND1 = NS1 * CAND


def _sc_retrieve_body(sims_hbm, vals_hbm, out_hbm,
                      chunk, cvs, cis, shv, shi, allv, alli,
                      idx_scr, rows, ret_scr, sem):
    s = lax.axis_index("s")
    base = s * ROWS1
    lane = jnp.arange(LANES, dtype=jnp.int32)
    negs = jnp.full((LANES,), NEG, dtype=jnp.float32)
    bigs = jnp.full((LANES,), BIGI, dtype=jnp.int32)

    pltpu.sync_copy(sims_hbm.at[pl.ds(base, ROWS1)], chunk)

    def _ins_fast(v, gidx, carry):
        b1, b2, b3, i1, i2, i3 = carry
        gt1 = v > b1
        nb1 = jnp.where(gt1, v, b1)
        ni1 = jnp.where(gt1, gidx, i1)
        d2 = jnp.where(gt1, b1, v)
        di2 = jnp.where(gt1, i1, gidx)
        gt2 = d2 > b2
        nb2 = jnp.where(gt2, d2, b2)
        ni2 = jnp.where(gt2, di2, i2)
        d3 = jnp.where(gt2, b2, d2)
        di3 = jnp.where(gt2, i2, di2)
        gt3 = d3 > b3
        nb3 = jnp.where(gt3, d3, b3)
        ni3 = jnp.where(gt3, di3, i3)
        return (nb1, nb2, nb3, ni1, ni2, ni3)

    def body(j, carry):
        for u in range(2):
            v = chunk[pl.ds((2 * j + u) * LANES, LANES)]
            gidx = lane + (base + (2 * j + u) * LANES)
            carry = _ins_fast(v, gidx, carry)
        return carry

    b1, b2, b3, i1, i2, i3 = lax.fori_loop(
        0, VECS1 // 2, body, (negs, negs, negs, bigs, bigs, bigs))
    cvs[pl.ds(0, LANES)] = b1
    cvs[pl.ds(LANES, LANES)] = b2
    cvs[pl.ds(2 * LANES, LANES)] = b3
    cis[pl.ds(0, LANES)] = i1
    cis[pl.ds(LANES, LANES)] = i2
    cis[pl.ds(2 * LANES, LANES)] = i3
    pltpu.sync_copy(cvs, shv.at[pl.ds(s * CAND, CAND)])
    pltpu.sync_copy(cis, shi.at[pl.ds(s * CAND, CAND)])
    plsc.subcore_barrier()

    pltpu.sync_copy(shv, allv)
    pltpu.sync_copy(shi, alli)

    def fold(k, carry):
        v = allv[pl.ds(k * LANES, LANES)]
        ii = alli[pl.ds(k * LANES, LANES)]
        return _insert3(v, ii, carry)

    carry = lax.fori_loop(0, NCAND1 // LANES, fold,
                          (negs, negs, negs, bigs, bigs, bigs))
    for d in (1, 2, 4, 8):
        perm = jnp.bitwise_xor(lane, d)
        b1, b2, b3, i1, i2, i3 = carry
        sb = [_permute(b, perm) for b in (b1, b2, b3)]
        si = [_permute(i, perm) for i in (i1, i2, i3)]
        for r in range(TOP_K):
            carry = _insert3(sb[r], si[r], carry)
    b1, b2, b3, i1, i2, i3 = carry

    e2 = jnp.exp(b2 - b1)
    e3 = jnp.exp(b3 - b1)
    denom = 1.0 + e2 + e3
    a0 = 1.0 / denom
    a1 = e2 / denom
    a2 = e3 / denom

    iv = jnp.where(lane == 0, i1, jnp.where(lane == 1, i2,
                   jnp.where(lane == 2, i3, 0)))
    idx_scr[...] = iv
    pltpu.async_copy(vals_hbm.at[idx_scr], rows, sem).wait()

    for cnk in range(MEM_DIM // LANES):
        ds = pl.ds(cnk * LANES, LANES)
        ret_scr[ds] = (a0 * rows[0, ds] + a1 * rows[1, ds]
                       + a2 * rows[2, ds])

    @pl.when(s == 0)
    def _():
        pltpu.sync_copy(ret_scr, out_hbm)


def _sc_retrieve(sims, mem_vals):
    mesh = plsc.VectorSubcoreMesh(core_axis_name="c", subcore_axis_name="s",
                                  num_cores=1)
    fn = functools.partial(
        pl.kernel, mesh=mesh,
        out_type=jax.ShapeDtypeStruct((MEM_DIM,), jnp.float32),
        scratch_types=[pltpu.VMEM((ROWS1,), jnp.float32),
                       pltpu.VMEM((CAND,), jnp.float32),
                       pltpu.VMEM((CAND,), jnp.int32),
                       pltpu.VMEM_SHARED((NCAND1,), jnp.float32),
                       pltpu.VMEM_SHARED((NCAND1,), jnp.int32),
                       pltpu.VMEM((NCAND1,), jnp.float32),
                       pltpu.VMEM((NCAND1,), jnp.int32),
                       pltpu.VMEM((LANES,), jnp.int32),
                       pltpu.VMEM((LANES, MEM_DIM), jnp.float32),
                       pltpu.VMEM((MEM_DIM,), jnp.float32),
                       pltpu.SemaphoreType.DMA],
    )(_sc_retrieve_body)
    return fn(sims, mem_vals)


def _tc2_body(p_ref, r_ref, Wb_ref, o_ref):
    o_ref[...] = p_ref[...] + jnp.dot(
        r_ref[...], Wb_ref[...], preferred_element_type=jnp.float32)


def _tc2(pout, ret, Wb):
    return pl.pallas_call(
        _tc2_body,
        out_shape=jax.ShapeDtypeStruct((1, OUT_SIZE), jnp.float32),
    )(pout, ret, Wb)


@jax.jit
def kernel(x, W1, b1, W2, b2, Wq, bq, mem_keys, mem_vals, importance, Wout, bout):
    x_last = x[:, -1, :]
    Wt = Wout[:HID]
    Wb = Wout[HID:]
    sims3, pout = _tc1(x_last, W1, b1.reshape(1, HID), W2, b2.reshape(1, HID),
                       Wq, bq.reshape(1, MEM_DIM), Wt, bout.reshape(1, OUT_SIZE),
                       mem_keys, importance)
    sims = sims3
    ret = _sc_retrieve(sims, mem_vals)
    return _tc2(pout, ret.reshape(1, MEM_DIM), Wb)


# speedup vs baseline: 1.0284x; 1.0284x over previous
import functools

import jax
import jax.numpy as jnp
from jax import lax
from jax.experimental import pallas as pl
from jax.experimental.pallas import tpu as pltpu
from jax.experimental.pallas import tpu_sc as plsc

IN_SIZE = 1024
HID = 1024
MEM_SIZE = 65536
MEM_DIM = 512
OUT_SIZE = 1024
TOP_K = 3

NUM_BLOCKS = 16
BLK = MEM_SIZE // NUM_BLOCKS

LANES = 16
CAND = TOP_K * LANES
NEG = -3.0e38
BIGI = 2**30


def _tc1_body(x_ref, W1_ref, b1_ref, W2_ref, b2_ref, Wq_ref, bq_ref,
              Wt_ref, bout_ref, mk_ref, imp_ref, sims_ref, pout_ref, qn_scr):
    i = pl.program_id(0)

    @pl.when(i == 0)
    def _():
        h1 = jnp.maximum(
            jnp.dot(x_ref[...], W1_ref[...], preferred_element_type=jnp.float32)
            + b1_ref[...], 0.0)
        h2 = jnp.dot(h1, W2_ref[...], preferred_element_type=jnp.float32) + b2_ref[...]
        q = jnp.dot(h2, Wq_ref[...], preferred_element_type=jnp.float32) + bq_ref[...]
        qn = q * (1.0 / jnp.maximum(jnp.sqrt(jnp.sum(q * q)), 1e-12))
        qn_scr[...] = qn
        pout_ref[...] = (jnp.dot(h2, Wt_ref[...], preferred_element_type=jnp.float32)
                         + bout_ref[...])

    mk = mk_ref[...]
    qn = qn_scr[...]
    dn = (((1,), (1,)), ((), ()))
    s = lax.dot_general(qn, mk, dn, preferred_element_type=jnp.float32)
    ones = jnp.ones((1, MEM_DIM), dtype=jnp.bfloat16)
    mk2 = (mk * mk).astype(jnp.bfloat16)
    rsq = lax.dot_general(ones, mk2, dn, preferred_element_type=jnp.float32)
    rn = 1.0 / jnp.maximum(jnp.sqrt(rsq), 1e-12)
    sims_ref[...] = (s * rn * imp_ref[...].reshape(1, BLK)).reshape(BLK)


def _tc1(x_last, W1, b1, W2, b2, Wq, bq, Wt, bout, mem_keys, imp):
    full2 = lambda shape: pl.BlockSpec(shape, lambda i: (0, 0))
    return pl.pallas_call(
        _tc1_body,
        grid=(NUM_BLOCKS,),
        in_specs=[
            full2((1, IN_SIZE)),
            full2((IN_SIZE, HID)),
            full2((1, HID)),
            full2((HID, HID)),
            full2((1, HID)),
            full2((HID, MEM_DIM)),
            full2((1, MEM_DIM)),
            full2((HID, OUT_SIZE)),
            full2((1, OUT_SIZE)),
            pl.BlockSpec((BLK, MEM_DIM), lambda i: (i, 0)),
            pl.BlockSpec((BLK,), lambda i: (i,)),
        ],
        out_specs=[
            pl.BlockSpec((BLK,), lambda i: (i,)),
            full2((1, OUT_SIZE)),
        ],
        out_shape=[
            jax.ShapeDtypeStruct((MEM_SIZE,), jnp.float32),
            jax.ShapeDtypeStruct((1, OUT_SIZE), jnp.float32),
        ],
        scratch_shapes=[pltpu.VMEM((1, MEM_DIM), jnp.float32)],
    )(x_last, W1, b1, W2, b2, Wq, bq, Wt, bout, mem_keys, imp)


def _tiecmp(v, i, mv, mi):
    return jnp.logical_or(v > mv, jnp.logical_and(v == mv, i < mi))


def _insert3(v, ii, carry):
    b1, b2, b3, i1, i2, i3 = carry
    c1 = _tiecmp(v, ii, b1, i1)
    nb1 = jnp.where(c1, v, b1)
    ni1 = jnp.where(c1, ii, i1)
    dv = jnp.where(c1, b1, v)
    di = jnp.where(c1, i1, ii)
    c2 = _tiecmp(dv, di, b2, i2)
    nb2 = jnp.where(c2, dv, b2)
    ni2 = jnp.where(c2, di, i2)
    dv2 = jnp.where(c2, b2, dv)
    di2 = jnp.where(c2, i2, di)
    c3 = _tiecmp(dv2, di2, b3, i3)
    nb3 = jnp.where(c3, dv2, b3)
    ni3 = jnp.where(c3, di2, i3)
    return (nb1, nb2, nb3, ni1, ni2, ni3)


def _permute(v, idx):
    dn = lax.GatherDimensionNumbers(
        offset_dims=(), collapsed_slice_dims=(0,), start_index_map=(0,))
    return lax.gather(v, idx[:, None], dn, slice_sizes=(1,),
                      mode=lax.GatherScatterMode.PROMISE_IN_BOUNDS)


NS1 = 16
ROWS1 = MEM_SIZE // NS1
VECS1 = ROWS1 // LANES
NCAND1 = NS1 * CAND


def _sc_retrieve_body(sims_hbm, vals_hbm, out_hbm,
                      chunk, cvs, cis, shv, shi, allv, alli,
                      idx_scr, rows, ret_scr, sem):
    s = lax.axis_index("s")
    base = s * ROWS1
    lane = jnp.arange(LANES, dtype=jnp.int32)
    negs = jnp.full((LANES,), NEG, dtype=jnp.float32)
    bigs = jnp.full((LANES,), BIGI, dtype=jnp.int32)

    pltpu.sync_copy(sims_hbm.at[pl.ds(base, ROWS1)], chunk)

    def _ins_fast(v, gidx, carry):
        b1, b2, b3, i1, i2, i3 = carry
        gt1 = v > b1
        nb1 = jnp.where(gt1, v, b1)
        ni1 = jnp.where(gt1, gidx, i1)
        d2 = jnp.where(gt1, b1, v)
        di2 = jnp.where(gt1, i1, gidx)
        gt2 = d2 > b2
        nb2 = jnp.where(gt2, d2, b2)
        ni2 = jnp.where(gt2, di2, i2)
        d3 = jnp.where(gt2, b2, d2)
        di3 = jnp.where(gt2, i2, di2)
        gt3 = d3 > b3
        nb3 = jnp.where(gt3, d3, b3)
        ni3 = jnp.where(gt3, di3, i3)
        return (nb1, nb2, nb3, ni1, ni2, ni3)

    def body(j, carry):
        for u in range(2):
            v = chunk[pl.ds((2 * j + u) * LANES, LANES)]
            gidx = lane + (base + (2 * j + u) * LANES)
            carry = _ins_fast(v, gidx, carry)
        return carry

    b1, b2, b3, i1, i2, i3 = lax.fori_loop(
        0, VECS1 // 2, body, (negs, negs, negs, bigs, bigs, bigs))
    cvs[pl.ds(0, LANES)] = b1
    cvs[pl.ds(LANES, LANES)] = b2
    cvs[pl.ds(2 * LANES, LANES)] = b3
    cis[pl.ds(0, LANES)] = i1
    cis[pl.ds(LANES, LANES)] = i2
    cis[pl.ds(2 * LANES, LANES)] = i3
    pltpu.sync_copy(cvs, shv.at[pl.ds(s * CAND, CAND)])
    pltpu.sync_copy(cis, shi.at[pl.ds(s * CAND, CAND)])
    plsc.subcore_barrier()

    pltpu.sync_copy(shv, allv)
    pltpu.sync_copy(shi, alli)

    def fold(k, carry):
        v = allv[pl.ds(k * LANES, LANES)]
        ii = alli[pl.ds(k * LANES, LANES)]
        return _insert3(v, ii, carry)

    carry = lax.fori_loop(0, NCAND1 // LANES, fold,
                          (negs, negs, negs, bigs, bigs, bigs))
    for d in (1, 2, 4, 8):
        perm = jnp.bitwise_xor(lane, d)
        b1, b2, b3, i1, i2, i3 = carry
        sb = [_permute(b, perm) for b in (b1, b2, b3)]
        si = [_permute(i, perm) for i in (i1, i2, i3)]
        for r in range(TOP_K):
            carry = _insert3(sb[r], si[r], carry)
    b1, b2, b3, i1, i2, i3 = carry

    e2 = jnp.exp(b2 - b1)
    e3 = jnp.exp(b3 - b1)
    denom = 1.0 + e2 + e3
    a0 = 1.0 / denom
    a1 = e2 / denom
    a2 = e3 / denom

    iv = jnp.where(lane == 0, i1, jnp.where(lane == 1, i2,
                   jnp.where(lane == 2, i3, 0)))
    idx_scr[...] = iv
    pltpu.async_copy(vals_hbm.at[idx_scr], rows, sem).wait()

    for cnk in range(MEM_DIM // LANES):
        ds = pl.ds(cnk * LANES, LANES)
        ret_scr[ds] = (a0 * rows[0, ds] + a1 * rows[1, ds]
                       + a2 * rows[2, ds])

    @pl.when(s == 0)
    def _():
        pltpu.sync_copy(ret_scr, out_hbm)


def _sc_retrieve(sims, mem_vals):
    mesh = plsc.VectorSubcoreMesh(core_axis_name="c", subcore_axis_name="s",
                                  num_cores=1)
    fn = functools.partial(
        pl.kernel, mesh=mesh,
        out_type=jax.ShapeDtypeStruct((MEM_DIM,), jnp.float32),
        scratch_types=[pltpu.VMEM((ROWS1,), jnp.float32),
                       pltpu.VMEM((CAND,), jnp.float32),
                       pltpu.VMEM((CAND,), jnp.int32),
                       pltpu.VMEM_SHARED((NCAND1,), jnp.float32),
                       pltpu.VMEM_SHARED((NCAND1,), jnp.int32),
                       pltpu.VMEM((NCAND1,), jnp.float32),
                       pltpu.VMEM((NCAND1,), jnp.int32),
                       pltpu.VMEM((LANES,), jnp.int32),
                       pltpu.VMEM((LANES, MEM_DIM), jnp.float32),
                       pltpu.VMEM((MEM_DIM,), jnp.float32),
                       pltpu.SemaphoreType.DMA],
    )(_sc_retrieve_body)
    return fn(sims, mem_vals)


def _tc2_body(p_ref, r_ref, Wb_ref, o_ref):
    o_ref[...] = p_ref[...] + jnp.dot(
        r_ref[...], Wb_ref[...], preferred_element_type=jnp.float32)


def _tc2(pout, ret, Wb):
    return pl.pallas_call(
        _tc2_body,
        out_shape=jax.ShapeDtypeStruct((1, OUT_SIZE), jnp.float32),
    )(pout, ret, Wb)


@jax.jit
def kernel(x, W1, b1, W2, b2, Wq, bq, mem_keys, mem_vals, importance, Wout, bout):
    x_last = x[:, -1, :]
    Wt = Wout[:HID]
    Wb = Wout[HID:]
    sims3, pout = _tc1(x_last, W1, b1.reshape(1, HID), W2, b2.reshape(1, HID),
                       Wq, bq.reshape(1, MEM_DIM), Wt, bout.reshape(1, OUT_SIZE),
                       mem_keys, importance)
    sims = sims3
    ret = _sc_retrieve(sims, mem_vals)
    return _tc2(pout, ret.reshape(1, MEM_DIM), Wb)
